# baseline (speedup 1.0000x reference)
import functools

import jax
import jax.numpy as jnp
from jax import lax
from jax.experimental import pallas as pl
from jax.experimental.pallas import tpu as pltpu
from jax.experimental.pallas import tpu_sc as plsc

NC = 2
NS = 16
NW = NC * NS
LANES = 128
K = 5


@functools.lru_cache(maxsize=None)
def _build(n_rows, vocab, dim):
    assert n_rows % (NW * LANES) == 0
    per_w = n_rows // NW
    groups = per_w // LANES
    assert groups % (2 * K) == 0
    nb = groups // K

    mesh = plsc.VectorSubcoreMesh(core_axis_name="c", subcore_axis_name="s")

    @functools.partial(
        pl.kernel,
        mesh=mesh,
        out_type=jax.ShapeDtypeStruct((n_rows, dim), jnp.float32),
        compiler_params=pltpu.CompilerParams(use_tc_tiling_on_sc=False),
        scratch_types=[
            pltpu.VMEM((groups, LANES), jnp.int32),
            pltpu.VMEM((K, LANES, dim), jnp.float32),
            pltpu.VMEM((K, LANES, dim), jnp.float32),
            pltpu.SemaphoreType.DMA,
            pltpu.SemaphoreType.DMA,
        ],
    )
    def emb(ids_hbm, table_hbm, out_hbm, idx_v, buf_a, buf_b, sem_a, sem_b):
        wid = lax.axis_index("s") * NC + lax.axis_index("c")
        gbase = wid * groups
        rbase = wid * per_w

        pltpu.sync_copy(ids_hbm.at[pl.ds(gbase, groups)], idx_v)

        def fire(block, buf, sem):
            for i in range(K):
                g = block * K + i
                pltpu.make_async_copy(
                    table_hbm.at[idx_v.at[g]], buf.at[i], sem
                ).start()

        def drain_flush(block, buf, sem):
            for i in range(K):
                pltpu.make_async_copy(
                    table_hbm.at[idx_v.at[i]], buf.at[i], sem
                ).wait()
            for i in range(K):
                g = block * K + i
                pltpu.sync_copy(
                    buf.at[i], out_hbm.at[pl.ds(rbase + g * LANES, LANES)]
                )

        fire(0, buf_a, sem_a)

        def body(t, carry):
            fire(2 * t + 1, buf_b, sem_b)
            drain_flush(2 * t, buf_a, sem_a)
            fire(2 * t + 2, buf_a, sem_a)
            drain_flush(2 * t + 1, buf_b, sem_b)
            return carry

        lax.fori_loop(0, nb // 2 - 1, body, 0)

        t_last = nb // 2 - 1
        fire(2 * t_last + 1, buf_b, sem_b)
        drain_flush(2 * t_last, buf_a, sem_a)
        drain_flush(2 * t_last + 1, buf_b, sem_b)

    return emb


def kernel(word_ids, word_emb_table):
    batch, seq = word_ids.shape
    vocab, dim = word_emb_table.shape
    n_rows = batch * seq
    ids_flat = word_ids.astype(jnp.int32).reshape(n_rows // LANES, LANES)
    emb = _build(n_rows, vocab, dim)
    out = emb(ids_flat, word_emb_table)
    return out.reshape(batch, seq, dim)
